# baseline (speedup 1.0000x reference)
import functools

import jax
import jax.numpy as jnp
from jax import lax
from jax.experimental import pallas as pl
from jax.experimental.pallas import tpu as pltpu
from jax.experimental.pallas import tpu_sc as plsc

N_BATCH = 16384
N_COLS = 200
LANES = 128
KB = N_BATCH // LANES
N_OUT = N_BATCH * N_COLS
N_IN = 2 * N_OUT
N_BLOCKS = N_COLS * KB

NC = 2
NS = 16
NW = NC * NS

SC_BLOCKS = 15360
TC_BLOCKS = N_BLOCKS - SC_BLOCKS
SC_OUT = SC_BLOCKS * LANES
PER_W_BLOCKS = SC_BLOCKS // NW

C_BLOCKS = 160
C_IN = C_BLOCKS * 2 * LANES
C_OUT = C_BLOCKS * LANES
N_CHUNKS = PER_W_BLOCKS // C_BLOCKS

TC_ROWS_IN = 2 * TC_BLOCKS
TC_BLK_IN = 1024
TC_GRID = TC_ROWS_IN // TC_BLK_IN

LN2 = 0.6931471805599453


def _log16(p):
    xi = plsc.bitcast(p, jnp.int32)
    e = ((xi >> 23) & 0xFF) - 127
    mi = (xi & 0x007FFFFF) | 0x3F800000
    m = plsc.bitcast(mi, jnp.float32)
    big = m > 1.4142135623730951
    m = jnp.where(big, m * 0.5, m)
    e = e + jnp.where(big, 1, 0)
    t = m - 1.0
    s = t / (2.0 + t)
    u = s * s
    log1p = 2.0 * s * (1.0 + u * (1.0 / 3.0 + u * (0.2 + u * (1.0 / 7.0
                                                              + u / 9.0))))
    return e.astype(jnp.float32) * LN2 + log1p


def _sc_body(z_hbm, prob_hbm, out_hbm,
             zc0, zc1, oc0, oc1, lpv,
             isem0, isem1, osem0, osem1):
    zc = [zc0, zc1]
    oc = [oc0, oc1]
    isem = [isem0, isem1]
    osem = [osem0, osem1]

    wid = lax.axis_index("s") * NC + lax.axis_index("c")
    in_base = wid * (PER_W_BLOCKS * 2 * LANES)
    out_base = wid * (PER_W_BLOCKS * LANES)

    cp_in = [None, None]
    cp_out = [None, None]
    cp_in[0] = pltpu.async_copy(
        z_hbm.at[pl.ds(in_base, C_IN)], zc[0], isem[0])

    pltpu.sync_copy(prob_hbm, lpv.at[pl.ds(0, 2)])
    vl = _log16(lpv[...])
    iota16 = lax.iota(jnp.int32, 16)
    l0 = jnp.sum(jnp.where(iota16 == 0, vl, 0.0))
    l1 = jnp.sum(jnp.where(iota16 == 1, vl, 0.0))
    v0 = jnp.zeros((16,), jnp.float32) + l0
    v1 = jnp.zeros((16,), jnp.float32) + l1

    for g in range(N_CHUNKS):
        b = g % 2
        if g + 1 < N_CHUNKS:
            nb = (g + 1) % 2
            cp_in[nb] = pltpu.async_copy(
                z_hbm.at[pl.ds(in_base + (g + 1) * C_IN, C_IN)],
                zc[nb], isem[nb])
        cp_in[b].wait()
        if cp_out[b] is not None:
            cp_out[b].wait()

        zcb = zc[b]
        ocb = oc[b]

        @plsc.parallel_loop(0, C_BLOCKS, unroll=4)
        def block_body(r):
            bi = r * (2 * LANES)
            bo = r * LANES
            for i in range(LANES // 16):
                a = zcb[pl.ds(bi + 16 * i, 16)]
                bv = zcb[pl.ds(bi + LANES + 16 * i, 16)]
                ocb[pl.ds(bo + 16 * i, 16)] = jnp.where(a != bv, v1, v0)

        cp_out[b] = pltpu.async_copy(
            ocb, out_hbm.at[pl.ds(out_base + g * C_OUT, C_OUT)], osem[b])

    cp_out[0].wait()
    cp_out[1].wait()


def _run_sc(z_flat, prob):
    mesh = plsc.VectorSubcoreMesh(core_axis_name="c", subcore_axis_name="s")
    f = functools.partial(
        pl.kernel,
        mesh=mesh,
        out_type=jax.ShapeDtypeStruct((N_OUT,), jnp.float32),
        scratch_types=[
            pltpu.VMEM((C_IN,), jnp.int32),
            pltpu.VMEM((C_IN,), jnp.int32),
            pltpu.VMEM((C_OUT,), jnp.float32),
            pltpu.VMEM((C_OUT,), jnp.float32),
            pltpu.VMEM((16,), jnp.float32),
            pltpu.SemaphoreType.DMA,
            pltpu.SemaphoreType.DMA,
            pltpu.SemaphoreType.DMA,
            pltpu.SemaphoreType.DMA,
        ],
        compiler_params=pltpu.CompilerParams(needs_layout_passes=False),
    )(_sc_body)
    return f(z_flat, prob)


def _tc_body(lp_ref, z_ref, o_ref):
    x = z_ref[...].reshape(TC_BLK_IN // 2, 2 * LANES)
    a = x[:, :LANES]
    bv = x[:, LANES:]
    o_ref[...] = jnp.where(a != bv, lp_ref[1], lp_ref[0])


def _run_tc(z2d, lp):
    row0 = 2 * SC_BLOCKS // TC_BLK_IN
    return pl.pallas_call(
        _tc_body,
        grid=(TC_GRID,),
        in_specs=[
            pl.BlockSpec(memory_space=pltpu.SMEM),
            pl.BlockSpec((TC_BLK_IN, LANES), lambda i: (row0 + i, 0)),
        ],
        out_specs=pl.BlockSpec((TC_BLK_IN // 2, LANES), lambda i: (i, 0)),
        out_shape=jax.ShapeDtypeStruct((TC_BLOCKS, LANES), jnp.float32),
    )(lp, z2d)


def kernel(z, prob):
    z_flat = (
        z.reshape(KB, LANES, N_COLS, 2)
        .transpose(2, 0, 3, 1)
        .reshape(N_IN)
    )
    out_sc = _run_sc(z_flat, prob)
    lp = jnp.log(prob)
    out_tc = _run_tc(z_flat.reshape(N_IN // LANES, LANES), lp)
    out_flat = lax.dynamic_update_slice(out_sc, out_tc.reshape(-1), (SC_OUT,))
    out = (
        out_flat.reshape(N_COLS, KB, LANES)
        .transpose(1, 2, 0)
        .reshape(N_BATCH, N_COLS, 1)
    )
    return out


# speedup vs baseline: 1.1291x; 1.1291x over previous
import functools

import jax
import jax.numpy as jnp
from jax import lax
from jax.experimental import pallas as pl
from jax.experimental.pallas import tpu as pltpu
from jax.experimental.pallas import tpu_sc as plsc

N_BATCH = 16384
N_COLS = 200
LANES = 128
KB = N_BATCH // LANES
N_OUT = N_BATCH * N_COLS
N_IN = 2 * N_OUT
N_BLOCKS = N_COLS * KB

NC = 2
NS = 16
NW = NC * NS
PER_W_BLOCKS = N_BLOCKS // NW

C_BLOCKS = 160
C_IN = C_BLOCKS * 2 * LANES
C_OUT = C_BLOCKS * LANES
CHUNKS = (80, 160, 160, 160, 160, 80)
assert sum(CHUNKS) == PER_W_BLOCKS

LN2 = 0.6931471805599453


def _log16(p):
    xi = plsc.bitcast(p, jnp.int32)
    e = ((xi >> 23) & 0xFF) - 127
    mi = (xi & 0x007FFFFF) | 0x3F800000
    m = plsc.bitcast(mi, jnp.float32)
    big = m > 1.4142135623730951
    m = jnp.where(big, m * 0.5, m)
    e = e + jnp.where(big, 1, 0)
    t = m - 1.0
    s = t / (2.0 + t)
    u = s * s
    log1p = 2.0 * s * (1.0 + u * (1.0 / 3.0 + u * (0.2 + u * (1.0 / 7.0
                                                              + u / 9.0))))
    return e.astype(jnp.float32) * LN2 + log1p


def _sc_body(z_hbm, prob_hbm, out_hbm,
             zc0, zc1, oc0, oc1, lpv,
             isem0, isem1, osem0, osem1):
    zc = [zc0, zc1]
    oc = [oc0, oc1]
    isem = [isem0, isem1]
    osem = [osem0, osem1]

    wid = lax.axis_index("s") * NC + lax.axis_index("c")
    in_base = wid * (PER_W_BLOCKS * 2 * LANES)
    out_base = wid * (PER_W_BLOCKS * LANES)

    starts = [sum(CHUNKS[:g]) for g in range(len(CHUNKS))]
    cp_in = [None, None]
    cp_out = [None, None]
    cp_in[0] = pltpu.async_copy(
        z_hbm.at[pl.ds(in_base, CHUNKS[0] * 2 * LANES)],
        zc[0].at[pl.ds(0, CHUNKS[0] * 2 * LANES)], isem[0])

    pltpu.sync_copy(prob_hbm, lpv.at[pl.ds(0, 2)])
    vl = _log16(lpv[...])
    iota16 = lax.iota(jnp.int32, 16)
    l0 = jnp.sum(jnp.where(iota16 == 0, vl, 0.0))
    l1 = jnp.sum(jnp.where(iota16 == 1, vl, 0.0))
    v0 = jnp.zeros((16,), jnp.float32) + l0
    v1 = jnp.zeros((16,), jnp.float32) + l1

    for g, nblk in enumerate(CHUNKS):
        b = g % 2
        if g + 1 < len(CHUNKS):
            nb = (g + 1) % 2
            nnext = CHUNKS[g + 1] * 2 * LANES
            cp_in[nb] = pltpu.async_copy(
                z_hbm.at[pl.ds(in_base + starts[g + 1] * 2 * LANES, nnext)],
                zc[nb].at[pl.ds(0, nnext)], isem[nb])
        cp_in[b].wait()
        if cp_out[b] is not None:
            cp_out[b].wait()

        zcb = zc[b]
        ocb = oc[b]

        @plsc.parallel_loop(0, nblk, unroll=4)
        def block_body(r):
            bi = r * (2 * LANES)
            bo = r * LANES
            for i in range(LANES // 16):
                a = zcb[pl.ds(bi + 16 * i, 16)]
                bv = zcb[pl.ds(bi + LANES + 16 * i, 16)]
                ocb[pl.ds(bo + 16 * i, 16)] = jnp.where(a != bv, v1, v0)

        cp_out[b] = pltpu.async_copy(
            ocb.at[pl.ds(0, nblk * LANES)],
            out_hbm.at[pl.ds(out_base + starts[g] * LANES, nblk * LANES)],
            osem[b])

    cp_out[0].wait()
    cp_out[1].wait()


def _run(z_flat, prob):
    mesh = plsc.VectorSubcoreMesh(core_axis_name="c", subcore_axis_name="s")
    f = functools.partial(
        pl.kernel,
        mesh=mesh,
        out_type=jax.ShapeDtypeStruct((N_OUT,), jnp.float32),
        scratch_types=[
            pltpu.VMEM((C_IN,), jnp.int32),
            pltpu.VMEM((C_IN,), jnp.int32),
            pltpu.VMEM((C_OUT,), jnp.float32),
            pltpu.VMEM((C_OUT,), jnp.float32),
            pltpu.VMEM((16,), jnp.float32),
            pltpu.SemaphoreType.DMA,
            pltpu.SemaphoreType.DMA,
            pltpu.SemaphoreType.DMA,
            pltpu.SemaphoreType.DMA,
        ],
        compiler_params=pltpu.CompilerParams(needs_layout_passes=False),
    )(_sc_body)
    return f(z_flat, prob)


def kernel(z, prob):
    z_flat = (
        z.reshape(KB, LANES, N_COLS, 2)
        .transpose(2, 0, 3, 1)
        .reshape(N_IN)
    )
    out_flat = _run(z_flat, prob)
    out = (
        out_flat.reshape(N_COLS, KB, LANES)
        .transpose(1, 2, 0)
        .reshape(N_BATCH, N_COLS, 1)
    )
    return out
